# baseline (device time: 9209 ns/iter reference)
import jax
import jax.numpy as jnp
from jax import lax
from jax.experimental import pallas as pl
from jax.experimental.pallas import tpu as pltpu

X_SIZE = 2
N_COL_BLOCKS = 2
N_ROW_BLOCKS = 4


def kernel(x):
    m_per, n_per = x.shape
    m_total = m_per * X_SIZE
    block_m = m_per // N_ROW_BLOCKS
    block_n = n_per // N_COL_BLOCKS

    def body(x_ref, out_ref, acc_ref, recv_ref, send_sems, recv_sems):
        j = pl.program_id(0)
        i = pl.program_id(1)
        my_x = lax.axis_index("x")
        my_y = lax.axis_index("y")
        peer_x = 1 - my_x

        col = j * block_n

        @pl.when(i == 0)
        def _():
            acc_ref[:, pl.ds(col, block_n)] = jnp.sum(
                x_ref[...], axis=0, keepdims=True
            )

        @pl.when(i > 0)
        def _():
            acc_ref[:, pl.ds(col, block_n)] += jnp.sum(
                x_ref[...], axis=0, keepdims=True
            )

        def exchange(jj):
            return pltpu.make_async_remote_copy(
                src_ref=acc_ref.at[:, pl.ds(jj * block_n, block_n)],
                dst_ref=recv_ref.at[:, pl.ds(jj * block_n, block_n)],
                send_sem=send_sems.at[jj],
                recv_sem=recv_sems.at[jj],
                device_id=(peer_x, my_y),
                device_id_type=pl.DeviceIdType.MESH,
            )

        @pl.when((j == 0) & (i == N_ROW_BLOCKS - 1))
        def _():
            barrier_sem = pltpu.get_barrier_semaphore()
            pl.semaphore_signal(
                barrier_sem,
                inc=1,
                device_id=(peer_x, my_y),
                device_id_type=pl.DeviceIdType.MESH,
            )
            pl.semaphore_wait(barrier_sem, 1)
            exchange(0).start()

        @pl.when((j == N_COL_BLOCKS - 1) & (i == N_ROW_BLOCKS - 1))
        def _():
            exchange(1).start()
            r0 = exchange(0)
            r1 = exchange(1)
            r0.wait_send()
            r1.wait_send()
            r0.wait_recv()
            r1.wait_recv()
            out_ref[...] = (acc_ref[...] + recv_ref[...]) * (1.0 / m_total)

    return pl.pallas_call(
        body,
        grid=(N_COL_BLOCKS, N_ROW_BLOCKS),
        out_shape=jax.ShapeDtypeStruct((1, n_per), jnp.float32),
        in_specs=[
            pl.BlockSpec(
                (block_m, block_n), lambda j, i: (i, j), memory_space=pltpu.VMEM
            )
        ],
        out_specs=pl.BlockSpec((1, n_per), lambda j, i: (0, 0)),
        scratch_shapes=[
            pltpu.VMEM((1, n_per), jnp.float32),
            pltpu.VMEM((1, n_per), jnp.float32),
            pltpu.SemaphoreType.DMA((N_COL_BLOCKS,)),
            pltpu.SemaphoreType.DMA((N_COL_BLOCKS,)),
        ],
        compiler_params=pltpu.CompilerParams(collective_id=0),
    )(x)


# device time: 8314 ns/iter; 1.1076x vs baseline; 1.1076x over previous
import jax
import jax.numpy as jnp
from jax import lax
from jax.experimental import pallas as pl
from jax.experimental.pallas import tpu as pltpu

X_SIZE = 2
N_ROW_BLOCKS = 8


def kernel(x):
    m_per, n_per = x.shape
    m_total = m_per * X_SIZE
    block_m = m_per // N_ROW_BLOCKS

    def body(x_ref, out_ref, acc_ref, comm_ref, send_sem, recv_sem):
        i = pl.program_id(0)
        my_x = lax.axis_index("x")
        my_y = lax.axis_index("y")
        peer_x = 1 - my_x

        @pl.when(i == 0)
        def _():
            acc_ref[...] = jnp.sum(x_ref[...], axis=0, keepdims=True)

        @pl.when(i > 0)
        def _():
            acc_ref[...] += jnp.sum(x_ref[...], axis=0, keepdims=True)

        @pl.when(i == N_ROW_BLOCKS - 1)
        def _():
            barrier_sem = pltpu.get_barrier_semaphore()
            pl.semaphore_signal(
                barrier_sem,
                inc=1,
                device_id=(peer_x, my_y),
                device_id_type=pl.DeviceIdType.MESH,
            )
            pl.semaphore_wait(barrier_sem, 1)
            comm_ref[0, :, :] = acc_ref[...]
            rdma = pltpu.make_async_remote_copy(
                src_ref=comm_ref.at[0],
                dst_ref=comm_ref.at[1],
                send_sem=send_sem,
                recv_sem=recv_sem,
                device_id=(peer_x, my_y),
                device_id_type=pl.DeviceIdType.MESH,
            )
            rdma.start()
            rdma.wait()
            out_ref[:, :] = (comm_ref[0, :, :] + comm_ref[1, :, :]) * (
                1.0 / m_total
            )

    return pl.pallas_call(
        body,
        grid=(N_ROW_BLOCKS,),
        out_shape=jax.ShapeDtypeStruct((1, n_per), jnp.float32),
        in_specs=[pl.BlockSpec((block_m, n_per), lambda i: (i, 0))],
        out_specs=pl.BlockSpec((1, n_per), lambda i: (0, 0)),
        scratch_shapes=[
            pltpu.VMEM((1, n_per), jnp.float32),
            pltpu.VMEM((2, 1, n_per), jnp.float32),
            pltpu.SemaphoreType.DMA,
            pltpu.SemaphoreType.DMA,
        ],
        compiler_params=pltpu.CompilerParams(collective_id=0),
    )(x)
